# baseline (speedup 1.0000x reference)
import jax
import jax.numpy as jnp
from jax import lax
from jax.experimental import pallas as pl
from jax.experimental.pallas import tpu as pltpu
from jax.experimental.pallas import tpu_sc as plsc

NUM_EMB = 1000
D = 64
B = 4096
L = 200

_INFO = plsc.get_sparse_core_info()
NC = _INFO.num_cores
NS = _INFO.num_subcores
NW = NC * NS
ROWS_PER_W = B // NW
CHUNK = ROWS_PER_W * L
TROWS = 208
PSTR = 129
NSLOT = 2
NRND = L // (2 * NSLOT)


def _body(cm_hbm, w_hbm, out_hbm, cm_v, pos_t, tab_v, stage_v, *sems):
    wid = lax.axis_index("s") * NC + lax.axis_index("c")
    base = wid * CHUNK

    pltpu.sync_copy(w_hbm.at[pl.ds(0, TROWS)], tab_v)
    pltpu.sync_copy(cm_hbm.at[pl.ds(base, CHUNK)], cm_v.at[pl.ds(0, CHUNK)])

    iota = lax.iota(jnp.int32, 16)
    lane_lt8 = iota < jnp.full((16,), 8, jnp.int32)
    ones = jnp.full((16,), 1, jnp.int32)
    n_chunks = (L + 15) // 16

    def row_body(r, _):
        carry = jnp.full((16,), 0, jnp.int32)
        row0 = r * L
        for j in range(n_chunks):
            off = row0 + j * 16
            last = j == n_chunks - 1
            x = cm_v[pl.ds(off, 16)]
            m = (x == ones).astype(jnp.int32)
            if last:
                m = m * lane_lt8.astype(jnp.int32)
            c = plsc.cumsum(m)
            pos = (c + carry) * m
            carry = carry + jnp.full((16,), jnp.sum(m), jnp.int32)
            addr = (iota + j * 16) * PSTR + r
            if last:
                plsc.store_scatter(pos_t, [addr], pos, mask=lane_lt8)
            else:
                plsc.store_scatter(pos_t, [addr], pos)
        return 0

    lax.fori_loop(0, ROWS_PER_W, row_body, 0)

    passes = []
    for m in range(16):
        for n in range(4):
            c = ((iota + m) & 15) + 16 * n
            passes.append((c, c >> 3, (c & 7) * 128))

    def fill(l, ref):
        def k_body(k, _):
            pos = pos_t[pl.ds(l * PSTR + k * 16, 16)]
            blane = iota + k * 16
            for c, c_hi, c_lo128 in passes:
                vals = plsc.load_gather(tab_v, [pos, c])
                plsc.store_scatter(ref, [c_hi, c_lo128 + blane], vals)
            return 0

        lax.fori_loop(0, ROWS_PER_W // 16, k_body, 0)

    def fill2(p, s):
        fill(2 * p, stage_v.at[s, 0])
        fill(2 * p + 1, stage_v.at[s, 1])

    def dst(p):
        return out_hbm.at[pl.ds(2 * p, 2), :, wid]

    for s in range(NSLOT):
        fill2(s, s)
        pltpu.async_copy(stage_v.at[s], dst(s), sems[s])

    def round_body(rd, _):
        for s in range(NSLOT):
            p = rd * NSLOT + s
            pltpu.make_async_copy(
                stage_v.at[s], dst(p - NSLOT), sems[s]
            ).wait()
            fill2(p, s)
            pltpu.async_copy(stage_v.at[s], dst(p), sems[s])
        return 0

    lax.fori_loop(1, NRND, round_body, 0)

    for s in range(NSLOT):
        pltpu.make_async_copy(
            stage_v.at[s], dst(L // 2 - NSLOT + s), sems[s]
        ).wait()


def _kernel_impl(chain_mask, weight):
    cm1d = chain_mask.reshape(B * L)
    run = pl.kernel(
        _body,
        out_type=jax.ShapeDtypeStruct((L, D // 8, B // 128, 8 * 128),
                                      jnp.float32),
        mesh=plsc.VectorSubcoreMesh(core_axis_name="c", subcore_axis_name="s"),
        compiler_params=pltpu.CompilerParams(
            use_tc_tiling_on_sc=False, needs_layout_passes=False
        ),
        scratch_types=[
            pltpu.VMEM((CHUNK + 16,), jnp.int32),
            pltpu.VMEM((L * PSTR + 16,), jnp.int32),
            pltpu.VMEM((TROWS, D), jnp.float32),
            pltpu.VMEM((NSLOT, 2, D // 8, 8 * 128), jnp.float32),
        ] + [pltpu.SemaphoreType.DMA] * NSLOT,
    )
    out = run(cm1d, weight).reshape(L, D // 8, B // 128, 8, 128)
    return out.transpose(2, 4, 0, 1, 3).reshape(B, L, D)


kernel = jax.jit(_kernel_impl)


# speedup vs baseline: 1.0922x; 1.0922x over previous
import jax
import jax.numpy as jnp
from jax import lax
from jax.experimental import pallas as pl
from jax.experimental.pallas import tpu as pltpu
from jax.experimental.pallas import tpu_sc as plsc

NUM_EMB = 1000
D = 64
B = 4096
L = 200

_INFO = plsc.get_sparse_core_info()
NC = _INFO.num_cores
NS = _INFO.num_subcores
NW = NC * NS
ROWS_PER_W = B // NW
CHUNK = ROWS_PER_W * L
TROWS = 208
BP = 512
NBLK = CHUNK // BP
NRND = NBLK // 2


def _body(cm_hbm, w_hbm, out_hbm, cm_v, tab_v, stage_v, sem0, sem1):
    wid = lax.axis_index("s") * NC + lax.axis_index("c")
    base = wid * CHUNK
    sems = (sem0, sem1)

    pltpu.sync_copy(w_hbm.at[pl.ds(0, TROWS)], tab_v)
    pltpu.sync_copy(cm_hbm.at[pl.ds(base, CHUNK)], cm_v.at[pl.ds(0, CHUNK)])

    iota = lax.iota(jnp.int32, 16)
    lane_lt8 = iota < jnp.full((16,), 8, jnp.int32)
    ones = jnp.full((16,), 1, jnp.int32)
    n_chunks = (L + 15) // 16

    def row_body(r, _):
        carry = jnp.full((16,), 0, jnp.int32)
        row0 = r * L
        for j in range(n_chunks):
            off = row0 + j * 16
            last = j == n_chunks - 1
            x = cm_v[pl.ds(off, 16)]
            m = (x == ones).astype(jnp.int32)
            if last:
                m = m * lane_lt8.astype(jnp.int32)
            c = plsc.cumsum(m)
            pos = (c + carry) * m
            carry = carry + jnp.full((16,), jnp.sum(m), jnp.int32)
            if last:
                pos = jnp.where(lane_lt8, pos, x)
            cm_v[pl.ds(off, 16)] = pos
        return 0

    lax.fori_loop(0, ROWS_PER_W, row_body, 0)

    cols = [
        ((iota + m) & 15) + 16 * n for m in range(16) for n in range(4)
    ]

    def fill_block(i, s):
        def t_body(t, _):
            pos = cm_v[pl.ds(i * BP + t * 16, 16)]
            row = iota + t * 16
            for c in cols:
                vals = plsc.load_gather(tab_v, [pos, c])
                plsc.store_scatter(stage_v.at[s], [row, c], vals)
            return 0

        lax.fori_loop(0, BP // 16, t_body, 0)

    def dst(i):
        return out_hbm.at[pl.ds(base + i * BP, BP), pl.ds(0, D)]

    for s in range(2):
        fill_block(s, s)
        pltpu.async_copy(stage_v.at[s], dst(s), sems[s])

    def round_body(r, _):
        for s in range(2):
            i = 2 * r + s
            pltpu.make_async_copy(stage_v.at[s], dst(i - 2), sems[s]).wait()
            fill_block(i, s)
            pltpu.async_copy(stage_v.at[s], dst(i), sems[s])
        return 0

    lax.fori_loop(1, NRND, round_body, 0)

    for s in range(2):
        pltpu.make_async_copy(
            stage_v.at[s], dst(NBLK - 2 + s), sems[s]
        ).wait()


def _kernel_impl(chain_mask, weight):
    cm1d = chain_mask.reshape(B * L)
    run = pl.kernel(
        _body,
        out_type=jax.ShapeDtypeStruct((B * L, 2 * D), jnp.float32),
        mesh=plsc.VectorSubcoreMesh(core_axis_name="c", subcore_axis_name="s"),
        compiler_params=pltpu.CompilerParams(
            use_tc_tiling_on_sc=False, needs_layout_passes=False
        ),
        scratch_types=[
            pltpu.VMEM((CHUNK + 16,), jnp.int32),
            pltpu.VMEM((TROWS, D), jnp.float32),
            pltpu.VMEM((2, BP, D), jnp.float32),
            pltpu.SemaphoreType.DMA,
            pltpu.SemaphoreType.DMA,
        ],
    )
    out = run(cm1d, weight).reshape(B, L, 2 * D)
    return out[:, :, :D]


kernel = jax.jit(_kernel_impl)
